# baseline (speedup 1.0000x reference)
import jax
import jax.numpy as jnp
from jax import lax
from jax.experimental import pallas as pl
from jax.experimental.pallas import tpu as pltpu
from jax.experimental.pallas import tpu_sc as plsc

N = 10000
FM = 128
H = 8
E = 320000
OUT_CH = 128
N_CIRC = 504

NP = 10240
NC, NS = 2, 16
NW = NC * NS
E_REAL = E + N
TA = 10752
EP = TA * NW
KA = 64
KB = 16
SBE = 256
NSB = TA // SBE
SB_B = SBE // KB
SB_A = SBE // KA
ZR = 32
ROWS_W = NP // NS

_MESH = plsc.VectorSubcoreMesh(core_axis_name="c", subcore_axis_name="s")


def _linear_body(xb, w_ref, asrc_ref, adst_ref, xw_ref, als_ref, ald_ref):
    xw = jnp.dot(xb, w_ref[...], preferred_element_type=jnp.float32)
    xw_ref[...] = xw
    row = lax.broadcasted_iota(jnp.int32, (H * FM, FM), 0) // FM
    col = lax.broadcasted_iota(jnp.int32, (H * FM, FM), 1)
    msk = (row == col).astype(jnp.float32)
    amat_s = jnp.reshape(asrc_ref[...], (H * FM, 1)) * msk
    amat_d = jnp.reshape(adst_ref[...], (H * FM, 1)) * msk
    als_ref[...] = jnp.dot(xw, amat_s, preferred_element_type=jnp.float32)
    ald_ref[...] = jnp.dot(xw, amat_d, preferred_element_type=jnp.float32)


def _linear1_kernel(x_ref, w_ref, asrc_ref, adst_ref, xw_ref, als_ref, ald_ref):
    _linear_body(x_ref[...], w_ref, asrc_ref, adst_ref, xw_ref, als_ref, ald_ref)


def _linear2_kernel(y0_ref, y1_ref, b_ref, w_ref, asrc_ref, adst_ref,
                    x_out_ref, xw_ref, als_ref, ald_ref):
    xb = jnp.maximum(y0_ref[...] + y1_ref[...] + b_ref[...], 0.0)
    x_out_ref[...] = xb
    _linear_body(xb, w_ref, asrc_ref, adst_ref, xw_ref, als_ref, ald_ref)


_BM = 512


def _tc_linear1(x_pad, W, a_src_f, a_dst_f):
    grid = NP // _BM
    return pl.pallas_call(
        _linear1_kernel,
        grid=(grid,),
        in_specs=[
            pl.BlockSpec((_BM, FM), lambda i: (i, 0)),
            pl.BlockSpec((FM, H * FM), lambda i: (0, 0)),
            pl.BlockSpec((1, H * FM), lambda i: (0, 0)),
            pl.BlockSpec((1, H * FM), lambda i: (0, 0)),
        ],
        out_specs=[
            pl.BlockSpec((_BM, H * FM), lambda i: (i, 0)),
            pl.BlockSpec((_BM, FM), lambda i: (i, 0)),
            pl.BlockSpec((_BM, FM), lambda i: (i, 0)),
        ],
        out_shape=[
            jax.ShapeDtypeStruct((NP, H * FM), jnp.float32),
            jax.ShapeDtypeStruct((NP, FM), jnp.float32),
            jax.ShapeDtypeStruct((NP, FM), jnp.float32),
        ],
    )(x_pad, W, a_src_f, a_dst_f)


def _tc_linear2(ypart, b, W, a_src_f, a_dst_f):
    grid = NP // _BM
    return pl.pallas_call(
        _linear2_kernel,
        grid=(grid,),
        in_specs=[
            pl.BlockSpec((_BM, FM), lambda i: (i, 0)),
            pl.BlockSpec((_BM, FM), lambda i: (i + NP // _BM, 0)),
            pl.BlockSpec((1, FM), lambda i: (0, 0)),
            pl.BlockSpec((FM, H * FM), lambda i: (0, 0)),
            pl.BlockSpec((1, H * FM), lambda i: (0, 0)),
            pl.BlockSpec((1, H * FM), lambda i: (0, 0)),
        ],
        out_specs=[
            pl.BlockSpec((_BM, FM), lambda i: (i, 0)),
            pl.BlockSpec((_BM, H * FM), lambda i: (i, 0)),
            pl.BlockSpec((_BM, FM), lambda i: (i, 0)),
            pl.BlockSpec((_BM, FM), lambda i: (i, 0)),
        ],
        out_shape=[
            jax.ShapeDtypeStruct((NP, FM), jnp.float32),
            jax.ShapeDtypeStruct((NP, H * FM), jnp.float32),
            jax.ShapeDtypeStruct((NP, FM), jnp.float32),
            jax.ShapeDtypeStruct((NP, FM), jnp.float32),
        ],
    )(ypart, ypart, b, W, a_src_f, a_dst_f)


def _dinv_kernel(d0_ref, d1_ref, out_ref):
    out_ref[...] = 1.0 / (H * (d0_ref[...] + d1_ref[...]) + H * 1e-16)


def _tc_dinv(denom_flat):
    blk = 1024
    return pl.pallas_call(
        _dinv_kernel,
        grid=(NP // blk,),
        in_specs=[
            pl.BlockSpec((blk, FM), lambda i: (i, 0)),
            pl.BlockSpec((blk, FM), lambda i: (i + NP // blk, 0)),
        ],
        out_specs=pl.BlockSpec((blk, FM), lambda i: (i, 0)),
        out_shape=jax.ShapeDtypeStruct((NP, FM), jnp.float32),
    )(denom_flat, denom_flat)


def _final_kernel(y0_ref, y1_ref, b2_ref, x1_ref, c1_ref, c2_ref, cb_ref, xo_ref):
    x2 = jnp.maximum(y0_ref[...] + y1_ref[...] + b2_ref[...], 0.0)
    xo_ref[...] = (
        jnp.dot(x1_ref[...], c1_ref[...], preferred_element_type=jnp.float32)
        + jnp.dot(x2, c2_ref[...], preferred_element_type=jnp.float32)
        + cb_ref[...]
    )


def _tc_final(ypart2, b2, x1, c1t, c2t, conv_b_row):
    grid = NP // _BM
    return pl.pallas_call(
        _final_kernel,
        grid=(grid,),
        in_specs=[
            pl.BlockSpec((_BM, FM), lambda i: (i, 0)),
            pl.BlockSpec((_BM, FM), lambda i: (i + NP // _BM, 0)),
            pl.BlockSpec((1, FM), lambda i: (0, 0)),
            pl.BlockSpec((_BM, FM), lambda i: (i, 0)),
            pl.BlockSpec((FM, OUT_CH), lambda i: (0, 0)),
            pl.BlockSpec((FM, OUT_CH), lambda i: (0, 0)),
            pl.BlockSpec((1, OUT_CH), lambda i: (0, 0)),
        ],
        out_specs=pl.BlockSpec((_BM, OUT_CH), lambda i: (i, 0)),
        out_shape=jax.ShapeDtypeStruct((NP, OUT_CH), jnp.float32),
    )(ypart2, ypart2, b2, x1, c1t, c2t, conv_b_row)


def _scores_kernel(a_ref, b_ref, out_ref):
    out_ref[...] = lax.dot_general(
        a_ref[...], b_ref[...], (((1,), (1,)), ((), ())),
        preferred_element_type=jnp.float32)


def _tc_scores(circ_pad, mirna_pad):
    bn = 640
    nb = mirna_pad.shape[0] // bn
    return pl.pallas_call(
        _scores_kernel,
        grid=(nb,),
        in_specs=[
            pl.BlockSpec((512, FM), lambda i: (0, 0)),
            pl.BlockSpec((bn, FM), lambda i: (i, 0)),
        ],
        out_specs=pl.BlockSpec((512, bn), lambda i: (0, i)),
        out_shape=jax.ShapeDtypeStruct((512, nb * bn), jnp.float32),
    )(circ_pad, mirna_pad)


def _zero_acc(acc_sh, zeros_hbm, zv, s):
    pltpu.sync_copy(zeros_hbm, zv)

    def z(i, _):
        pltpu.sync_copy(zv, acc_sh.at[pl.ds(s * ROWS_W + i * ZR, ZR)])
        return 0

    lax.fori_loop(0, ROWS_W // ZR, z, 0)


def _drain_acc(acc_sh, out_hbm, zv, c, s):
    def d(i, _):
        pltpu.sync_copy(acc_sh.at[pl.ds(s * ROWS_W + i * ZR, ZR)], zv)
        pltpu.sync_copy(zv, out_hbm.at[pl.ds(c * NP + s * ROWS_W + i * ZR, ZR)])
        return 0

    lax.fori_loop(0, ROWS_W // ZR, d, 0)


def _edge_att_body(src_hbm, dst_hbm, als_hbm, ald_hbm, zeros_hbm,
                   ex_out, denom_out,
                   sbs0, sbs1, sbd0, sbd1, sms0, sms1, smd0, smd1,
                   av0, av1, bv0, bv1, exl, exs, zv, denom_sh,
                   sem_a0, sem_a1, sem_b0, sem_b1):
    c = lax.axis_index("c")
    s = lax.axis_index("s")
    wid = s * NC + c
    _zero_acc(denom_sh, zeros_hbm, zv, s)
    pltpu.sync_copy(zeros_hbm, exs.at[pl.ds(0, ZR)])
    pltpu.sync_copy(zeros_hbm, exs.at[pl.ds(ZR, ZR)])
    plsc.subcore_barrier()

    base0 = wid * TA
    sb_idx = ((sbs0, sbd0), (sbs1, sbd1))
    sm = ((sms0, smd0), (sms1, smd1))
    gb = ((av0, bv0, sem_a0, sem_b0), (av1, bv1, sem_a1, sem_b1))

    def load_sb(p, sb):
        off = base0 + sb * SBE
        pltpu.sync_copy(src_hbm.at[pl.ds(off, SBE)], sb_idx[p][0])
        pltpu.sync_copy(dst_hbm.at[pl.ds(off, SBE)], sb_idx[p][1])

    def prep(p, j, b):
        for q in range(KA // 16):
            sm[b][0][pl.ds(q * 16, 16)] = sb_idx[p][0][pl.ds(j * KA + q * 16, 16)]
            sm[b][1][pl.ds(q * 16, 16)] = sb_idx[p][1][pl.ds(j * KA + q * 16, 16)]
        av, bv, sa, sbm = gb[b]
        pltpu.async_copy(als_hbm.at[sm[b][0]], av, sa)
        pltpu.async_copy(ald_hbm.at[sm[b][1]], bv, sbm)

    load_sb(0, 0)
    prep(0, 0, 0)

    def outer(g, _):
        for p in range(2):
            sb = 2 * g + p

            def inner(jj, _):
                for b in range(2):
                    j = jj * 2 + b
                    av, bv, sa, sbm = gb[b]
                    pltpu.make_async_copy(als_hbm.at[sm[b][0]], av, sa).wait()
                    pltpu.make_async_copy(ald_hbm.at[sm[b][1]], bv, sbm).wait()

                    @pl.when(j + 1 < SB_A)
                    def _():
                        prep(p, j + 1, 1 - b)

                    @pl.when(j + 1 == SB_A)
                    def _():
                        @pl.when(sb + 1 < NSB)
                        def _():
                            load_sb(1 - p, sb + 1)
                            prep(1 - p, 0, 1 - b)

                    def row(i, _):
                        a = av[i, pl.ds(0, 16)] + bv[i, pl.ds(0, 16)]
                        al = jnp.maximum(a, 0.2 * a)
                        e = jnp.exp(al)
                        exl[pl.ds(i * 16, 16)] = e
                        exs[i, pl.ds(0, 16)] = e
                        return 0

                    lax.fori_loop(0, KA, row, 0)
                    base = base0 + sb * SBE + j * KA
                    pltpu.sync_copy(exl, ex_out.at[pl.ds(base * 16, KA * 16)])
                    pltpu.sync_copy(exs, denom_sh.at[sm[b][1]], add=True)
                return 0

            lax.fori_loop(0, SB_A // 2, inner, 0)
        return 0

    lax.fori_loop(0, NSB // 2, outer, 0)
    plsc.subcore_barrier()
    _drain_acc(denom_sh, denom_out, zv, c, s)


def _sc_edge_att(src_all, dst_all, als, ald, zeros):
    k = pl.kernel(
        _edge_att_body,
        out_type=[
            jax.ShapeDtypeStruct((EP * 16,), jnp.float32),
            jax.ShapeDtypeStruct((NC * NP, FM), jnp.float32),
        ],
        mesh=_MESH,
        scratch_types=[
            pltpu.VMEM((SBE,), jnp.int32),
            pltpu.VMEM((SBE,), jnp.int32),
            pltpu.VMEM((SBE,), jnp.int32),
            pltpu.VMEM((SBE,), jnp.int32),
            pltpu.VMEM((KA,), jnp.int32),
            pltpu.VMEM((KA,), jnp.int32),
            pltpu.VMEM((KA,), jnp.int32),
            pltpu.VMEM((KA,), jnp.int32),
            pltpu.VMEM((KA, FM), jnp.float32),
            pltpu.VMEM((KA, FM), jnp.float32),
            pltpu.VMEM((KA, FM), jnp.float32),
            pltpu.VMEM((KA, FM), jnp.float32),
            pltpu.VMEM((KA * 16,), jnp.float32),
            pltpu.VMEM((KA, FM), jnp.float32),
            pltpu.VMEM((ZR, FM), jnp.float32),
            pltpu.VMEM_SHARED((NP, FM), jnp.float32),
            pltpu.SemaphoreType.DMA,
            pltpu.SemaphoreType.DMA,
            pltpu.SemaphoreType.DMA,
            pltpu.SemaphoreType.DMA,
        ],
    )
    return k(src_all, dst_all, als, ald, zeros)


def _edge_agg_body(src_hbm, dst_hbm, xw_hbm, ex_hbm, dinv_hbm, zeros_hbm,
                   y_out,
                   sbs0, sbs1, sbd0, sbd1, sms0, sms1, smd0, smd1,
                   xv0, xv1, dv0, dv1, exl_sb, cv, zv, y_sh,
                   sem_x0, sem_x1, sem_d0, sem_d1):
    c = lax.axis_index("c")
    s = lax.axis_index("s")
    wid = s * NC + c
    _zero_acc(y_sh, zeros_hbm, zv, s)
    plsc.subcore_barrier()

    base0 = wid * TA
    sb_idx = ((sbs0, sbd0), (sbs1, sbd1))
    sm = ((sms0, smd0), (sms1, smd1))
    gb = ((xv0, dv0, sem_x0, sem_d0), (xv1, dv1, sem_x1, sem_d1))

    def load_sb(p, sb):
        off = base0 + sb * SBE
        pltpu.sync_copy(src_hbm.at[pl.ds(off, SBE)], sb_idx[p][0])
        pltpu.sync_copy(dst_hbm.at[pl.ds(off, SBE)], sb_idx[p][1])

    def prep(p, j, b):
        sm[b][0][...] = sb_idx[p][0][pl.ds(j * KB, 16)]
        sm[b][1][...] = sb_idx[p][1][pl.ds(j * KB, 16)]
        xv, dv, sx, sd = gb[b]
        pltpu.async_copy(xw_hbm.at[sm[b][0]], xv, sx)
        pltpu.async_copy(dinv_hbm.at[sm[b][1]], dv, sd)

    load_sb(0, 0)
    prep(0, 0, 0)

    def outer(g, _):
        for p in range(2):
            sb = 2 * g + p
            pltpu.sync_copy(
                ex_hbm.at[pl.ds((base0 + sb * SBE) * 16, SBE * 16)], exl_sb)

            def inner(jj, _):
                for b in range(2):
                    j = jj * 2 + b
                    xv, dv, sx, sd = gb[b]
                    pltpu.make_async_copy(xw_hbm.at[sm[b][0]], xv, sx).wait()
                    pltpu.make_async_copy(dinv_hbm.at[sm[b][1]], dv, sd).wait()

                    @pl.when(j + 1 < SB_B)
                    def _():
                        prep(p, j + 1, 1 - b)

                    @pl.when(j + 1 == SB_B)
                    def _():
                        @pl.when(sb + 1 < NSB)
                        def _():
                            load_sb(1 - p, sb + 1)
                            prep(1 - p, 0, 1 - b)

                    def edge(i, _):
                        wrow = (exl_sb[pl.ds((j * KB + i) * 16, 16)]
                                * dv[i, pl.ds(0, 16)])
                        acc = [jnp.zeros((16,), jnp.float32)
                               for _ in range(FM // 16)]
                        for h in range(H):
                            w = jnp.full((16,), wrow[h])
                            for jf in range(FM // 16):
                                acc[jf] = acc[jf] + w * xv[
                                    i, pl.ds(h * FM + jf * 16, 16)]
                        for jf in range(FM // 16):
                            cv[i, pl.ds(jf * 16, 16)] = acc[jf]
                        return 0

                    lax.fori_loop(0, KB, edge, 0)
                    pltpu.sync_copy(cv, y_sh.at[sm[b][1]], add=True)
                return 0

            lax.fori_loop(0, SB_B // 2, inner, 0)
        return 0

    lax.fori_loop(0, NSB // 2, outer, 0)
    plsc.subcore_barrier()
    _drain_acc(y_sh, y_out, zv, c, s)


def _sc_edge_agg(src_all, dst_all, xw, ex, dinv, zeros):
    k = pl.kernel(
        _edge_agg_body,
        out_type=jax.ShapeDtypeStruct((NC * NP, FM), jnp.float32),
        mesh=_MESH,
        scratch_types=[
            pltpu.VMEM((SBE,), jnp.int32),
            pltpu.VMEM((SBE,), jnp.int32),
            pltpu.VMEM((SBE,), jnp.int32),
            pltpu.VMEM((SBE,), jnp.int32),
            pltpu.VMEM((KB,), jnp.int32),
            pltpu.VMEM((KB,), jnp.int32),
            pltpu.VMEM((KB,), jnp.int32),
            pltpu.VMEM((KB,), jnp.int32),
            pltpu.VMEM((KB, H * FM), jnp.float32),
            pltpu.VMEM((KB, H * FM), jnp.float32),
            pltpu.VMEM((KB, FM), jnp.float32),
            pltpu.VMEM((KB, FM), jnp.float32),
            pltpu.VMEM((SBE * 16,), jnp.float32),
            pltpu.VMEM((KB, FM), jnp.float32),
            pltpu.VMEM((ZR, FM), jnp.float32),
            pltpu.VMEM_SHARED((NP, FM), jnp.float32),
            pltpu.SemaphoreType.DMA,
            pltpu.SemaphoreType.DMA,
            pltpu.SemaphoreType.DMA,
            pltpu.SemaphoreType.DMA,
        ],
    )
    return k(src_all, dst_all, xw, ex, dinv, zeros)


def _gat_layer(src_all, dst_all, xw, als, ald, zeros):
    ex, denom = _sc_edge_att(src_all, dst_all, als, ald, zeros)
    dinv = _tc_dinv(denom)
    return _sc_edge_agg(src_all, dst_all, xw, ex, dinv, zeros)


def kernel(x, edge_index, W1, a_src1, a_dst1, b1, W2, a_src2, a_dst2, b2,
           conv_w, conv_b):
    x_pad = jnp.zeros((NP, FM), jnp.float32).at[:N].set(x)
    loops = jnp.arange(N, dtype=jnp.int32)
    dummy = jnp.full((EP - E_REAL,), N, dtype=jnp.int32)
    src_all = jnp.concatenate([edge_index[0], loops, dummy])
    dst_all = jnp.concatenate([edge_index[1], loops, dummy])
    a_src1_f = a_src1.reshape(1, H * FM)
    a_dst1_f = a_dst1.reshape(1, H * FM)
    a_src2_f = a_src2.reshape(1, H * FM)
    a_dst2_f = a_dst2.reshape(1, H * FM)
    b1_row = b1.reshape(1, FM)
    b2_row = b2.reshape(1, FM)
    c1t = conv_w[:, 0, :, 0].T
    c2t = conv_w[:, 1, :, 0].T
    conv_b_row = conv_b.reshape(1, OUT_CH)
    zeros = jnp.zeros((ZR, FM), jnp.float32)

    xw1, als1, ald1 = _tc_linear1(x_pad, W1, a_src1_f, a_dst1_f)
    y1part = _gat_layer(src_all, dst_all, xw1, als1, ald1, zeros)

    x1, xw2, als2, ald2 = _tc_linear2(y1part, b1_row, W2, a_src2_f, a_dst2_f)
    y2part = _gat_layer(src_all, dst_all, xw2, als2, ald2, zeros)

    xo = _tc_final(y2part, b2_row, x1, c1t, c2t, conv_b_row)
    circ_pad = xo[:512]
    mirna_pad = xo[N_CIRC:N_CIRC + 9600]
    scores_full = _tc_scores(circ_pad, mirna_pad)

    circ = xo[:N_CIRC]
    mirna = xo[N_CIRC:N]
    scores = scores_full[:N_CIRC, :N - N_CIRC]
    return scores, circ, mirna


# speedup vs baseline: 1.1695x; 1.1695x over previous
import jax
import jax.numpy as jnp
from jax import lax
from jax.experimental import pallas as pl
from jax.experimental.pallas import tpu as pltpu
from jax.experimental.pallas import tpu_sc as plsc

N = 10000
FM = 128
H = 8
E = 320000
OUT_CH = 128
N_CIRC = 504

NP = 10240
NC, NS = 2, 16
NW = NC * NS
E_REAL = E + N
TA = 10752
EP = TA * NW
KA = 64
KB = 16
SBE = 256
NSB = TA // SBE
SB_B = SBE // KB
SB_A = SBE // KA
ZR = 32
ROWS_W = NP // NS

_MESH = plsc.VectorSubcoreMesh(core_axis_name="c", subcore_axis_name="s")


def _linear_body(xb, w_ref, asrc_ref, adst_ref, xw_ref, als_ref, ald_ref):
    xw = jnp.dot(xb, w_ref[...], preferred_element_type=jnp.float32)
    xw_ref[...] = xw
    row = lax.broadcasted_iota(jnp.int32, (H * FM, FM), 0) // FM
    col = lax.broadcasted_iota(jnp.int32, (H * FM, FM), 1)
    msk = (row == col).astype(jnp.float32)
    amat_s = jnp.reshape(asrc_ref[...], (H * FM, 1)) * msk
    amat_d = jnp.reshape(adst_ref[...], (H * FM, 1)) * msk
    als_ref[...] = jnp.dot(xw, amat_s, preferred_element_type=jnp.float32)
    ald_ref[...] = jnp.dot(xw, amat_d, preferred_element_type=jnp.float32)


def _linear1_kernel(x_ref, w_ref, asrc_ref, adst_ref, xw_ref, als_ref, ald_ref):
    _linear_body(x_ref[...], w_ref, asrc_ref, adst_ref, xw_ref, als_ref, ald_ref)


def _linear2_kernel(y0_ref, y1_ref, b_ref, w_ref, asrc_ref, adst_ref,
                    x_out_ref, xw_ref, als_ref, ald_ref):
    xb = jnp.maximum(y0_ref[...] + y1_ref[...] + b_ref[...], 0.0)
    x_out_ref[...] = xb
    _linear_body(xb, w_ref, asrc_ref, adst_ref, xw_ref, als_ref, ald_ref)


_BM = 512


def _tc_linear1(x_pad, W, a_src_f, a_dst_f):
    grid = NP // _BM
    return pl.pallas_call(
        _linear1_kernel,
        grid=(grid,),
        in_specs=[
            pl.BlockSpec((_BM, FM), lambda i: (i, 0)),
            pl.BlockSpec((FM, H * FM), lambda i: (0, 0)),
            pl.BlockSpec((1, H * FM), lambda i: (0, 0)),
            pl.BlockSpec((1, H * FM), lambda i: (0, 0)),
        ],
        out_specs=[
            pl.BlockSpec((_BM, H * FM), lambda i: (i, 0)),
            pl.BlockSpec((_BM, FM), lambda i: (i, 0)),
            pl.BlockSpec((_BM, FM), lambda i: (i, 0)),
        ],
        out_shape=[
            jax.ShapeDtypeStruct((NP, H * FM), jnp.float32),
            jax.ShapeDtypeStruct((NP, FM), jnp.float32),
            jax.ShapeDtypeStruct((NP, FM), jnp.float32),
        ],
    )(x_pad, W, a_src_f, a_dst_f)


def _tc_linear2(ypart, b, W, a_src_f, a_dst_f):
    grid = NP // _BM
    return pl.pallas_call(
        _linear2_kernel,
        grid=(grid,),
        in_specs=[
            pl.BlockSpec((_BM, FM), lambda i: (i, 0)),
            pl.BlockSpec((_BM, FM), lambda i: (i + NP // _BM, 0)),
            pl.BlockSpec((1, FM), lambda i: (0, 0)),
            pl.BlockSpec((FM, H * FM), lambda i: (0, 0)),
            pl.BlockSpec((1, H * FM), lambda i: (0, 0)),
            pl.BlockSpec((1, H * FM), lambda i: (0, 0)),
        ],
        out_specs=[
            pl.BlockSpec((_BM, FM), lambda i: (i, 0)),
            pl.BlockSpec((_BM, H * FM), lambda i: (i, 0)),
            pl.BlockSpec((_BM, FM), lambda i: (i, 0)),
            pl.BlockSpec((_BM, FM), lambda i: (i, 0)),
        ],
        out_shape=[
            jax.ShapeDtypeStruct((NP, FM), jnp.float32),
            jax.ShapeDtypeStruct((NP, H * FM), jnp.float32),
            jax.ShapeDtypeStruct((NP, FM), jnp.float32),
            jax.ShapeDtypeStruct((NP, FM), jnp.float32),
        ],
    )(ypart, ypart, b, W, a_src_f, a_dst_f)


def _dinv_kernel(d0_ref, d1_ref, out_ref):
    out_ref[...] = 1.0 / (H * (d0_ref[...] + d1_ref[...]) + H * 1e-16)


def _tc_dinv(denom_flat):
    blk = 1024
    return pl.pallas_call(
        _dinv_kernel,
        grid=(NP // blk,),
        in_specs=[
            pl.BlockSpec((blk, FM), lambda i: (i, 0)),
            pl.BlockSpec((blk, FM), lambda i: (i + NP // blk, 0)),
        ],
        out_specs=pl.BlockSpec((blk, FM), lambda i: (i, 0)),
        out_shape=jax.ShapeDtypeStruct((NP, FM), jnp.float32),
    )(denom_flat, denom_flat)


def _final_kernel(y0_ref, y1_ref, b2_ref, x1_ref, c1_ref, c2_ref, cb_ref, xo_ref):
    x2 = jnp.maximum(y0_ref[...] + y1_ref[...] + b2_ref[...], 0.0)
    xo_ref[...] = (
        jnp.dot(x1_ref[...], c1_ref[...], preferred_element_type=jnp.float32)
        + jnp.dot(x2, c2_ref[...], preferred_element_type=jnp.float32)
        + cb_ref[...]
    )


def _tc_final(ypart2, b2, x1, c1t, c2t, conv_b_row):
    grid = NP // _BM
    return pl.pallas_call(
        _final_kernel,
        grid=(grid,),
        in_specs=[
            pl.BlockSpec((_BM, FM), lambda i: (i, 0)),
            pl.BlockSpec((_BM, FM), lambda i: (i + NP // _BM, 0)),
            pl.BlockSpec((1, FM), lambda i: (0, 0)),
            pl.BlockSpec((_BM, FM), lambda i: (i, 0)),
            pl.BlockSpec((FM, OUT_CH), lambda i: (0, 0)),
            pl.BlockSpec((FM, OUT_CH), lambda i: (0, 0)),
            pl.BlockSpec((1, OUT_CH), lambda i: (0, 0)),
        ],
        out_specs=pl.BlockSpec((_BM, OUT_CH), lambda i: (i, 0)),
        out_shape=jax.ShapeDtypeStruct((NP, OUT_CH), jnp.float32),
    )(ypart2, ypart2, b2, x1, c1t, c2t, conv_b_row)


def _scores_kernel(a_ref, b_ref, out_ref):
    out_ref[...] = lax.dot_general(
        a_ref[...], b_ref[...], (((1,), (1,)), ((), ())),
        preferred_element_type=jnp.float32)


def _tc_scores(circ_pad, mirna_pad):
    bn = 640
    nb = mirna_pad.shape[0] // bn
    return pl.pallas_call(
        _scores_kernel,
        grid=(nb,),
        in_specs=[
            pl.BlockSpec((512, FM), lambda i: (0, 0)),
            pl.BlockSpec((bn, FM), lambda i: (i, 0)),
        ],
        out_specs=pl.BlockSpec((512, bn), lambda i: (0, i)),
        out_shape=jax.ShapeDtypeStruct((512, nb * bn), jnp.float32),
    )(circ_pad, mirna_pad)


def _zero_acc(acc_sh, zeros_hbm, zv, s):
    pltpu.sync_copy(zeros_hbm, zv)

    def z(i, _):
        pltpu.sync_copy(zv, acc_sh.at[pl.ds(s * ROWS_W + i * ZR, ZR)])
        return 0

    lax.fori_loop(0, ROWS_W // ZR, z, 0)


def _drain_acc(acc_sh, out_hbm, zv, c, s):
    def d(i, _):
        pltpu.sync_copy(acc_sh.at[pl.ds(s * ROWS_W + i * ZR, ZR)], zv)
        pltpu.sync_copy(zv, out_hbm.at[pl.ds(c * NP + s * ROWS_W + i * ZR, ZR)])
        return 0

    lax.fori_loop(0, ROWS_W // ZR, d, 0)


def _edge_att_body(src_hbm, dst_hbm, als_hbm, ald_hbm, zeros_hbm,
                   ex_out, denom_out,
                   sbs0, sbd0, sms0, sms1, smd0, smd1,
                   av0, av1, bv0, bv1, exl, exs, zv, denom_sh,
                   sem_a0, sem_a1, sem_b0, sem_b1):
    c = lax.axis_index("c")
    s = lax.axis_index("s")
    wid = s * NC + c
    _zero_acc(denom_sh, zeros_hbm, zv, s)
    pltpu.sync_copy(zeros_hbm, exs.at[pl.ds(0, ZR)])
    pltpu.sync_copy(zeros_hbm, exs.at[pl.ds(ZR, ZR)])
    plsc.subcore_barrier()

    base0 = wid * TA
    sm = ((sms0, smd0), (sms1, smd1))
    gb = ((av0, bv0, sem_a0, sem_b0), (av1, bv1, sem_a1, sem_b1))

    def load_sb(sb):
        off = base0 + sb * SBE
        pltpu.sync_copy(src_hbm.at[pl.ds(off, SBE + KA)], sbs0)
        pltpu.sync_copy(dst_hbm.at[pl.ds(off, SBE + KA)], sbd0)

    def prep(j, b):
        for q in range(KA // 16):
            sm[b][0][pl.ds(q * 16, 16)] = sbs0[pl.ds(j * KA + q * 16, 16)]
            sm[b][1][pl.ds(q * 16, 16)] = sbd0[pl.ds(j * KA + q * 16, 16)]
        av, bv, sa, sbm = gb[b]
        pltpu.async_copy(als_hbm.at[sm[b][0]], av, sa)
        pltpu.async_copy(ald_hbm.at[sm[b][1]], bv, sbm)

    def step(sb, j, b, do_prep=True):
        av, bv, sa, sbm = gb[b]
        pltpu.make_async_copy(als_hbm.at[sm[b][0]], av, sa).wait()
        pltpu.make_async_copy(ald_hbm.at[sm[b][1]], bv, sbm).wait()
        if do_prep:
            prep(j + 1, 1 - b)

        def row(i, _):
            a = av[i, pl.ds(0, 16)] + bv[i, pl.ds(0, 16)]
            al = jnp.maximum(a, 0.2 * a)
            e = jnp.exp(al)
            exl[pl.ds(i * 16, 16)] = e
            exs[i, pl.ds(0, 16)] = e
            return 0

        lax.fori_loop(0, KA, row, 0)
        base = base0 + sb * SBE + j * KA
        pltpu.sync_copy(exl, ex_out.at[pl.ds(base * 16, KA * 16)])
        pltpu.sync_copy(exs, denom_sh.at[sm[b][1]], add=True)

    load_sb(0)
    prep(0, 0)

    def outer(sb, _):
        def inner(jj, _):
            for b in range(2):
                step(sb, 2 * jj + b, b)
            return 0

        lax.fori_loop(0, SB_A // 2, inner, 0)
        load_sb(sb + 1)
        return 0

    lax.fori_loop(0, NSB - 1, outer, 0)
    fsb = NSB - 1
    for jj in range(SB_A // 2 - 1):
        for b in range(2):
            step(fsb, 2 * jj + b, b)
    step(fsb, SB_A - 2, 0)
    step(fsb, SB_A - 1, 1, do_prep=False)
    plsc.subcore_barrier()
    _drain_acc(denom_sh, denom_out, zv, c, s)


def _sc_edge_att(src_all, dst_all, als, ald, zeros):
    k = pl.kernel(
        _edge_att_body,
        out_type=[
            jax.ShapeDtypeStruct((EP * 16,), jnp.float32),
            jax.ShapeDtypeStruct((NC * NP, FM), jnp.float32),
        ],
        mesh=_MESH,
        scratch_types=[
            pltpu.VMEM((SBE + KA,), jnp.int32),
            pltpu.VMEM((SBE + KA,), jnp.int32),
            pltpu.VMEM((KA,), jnp.int32),
            pltpu.VMEM((KA,), jnp.int32),
            pltpu.VMEM((KA,), jnp.int32),
            pltpu.VMEM((KA,), jnp.int32),
            pltpu.VMEM((KA, FM), jnp.float32),
            pltpu.VMEM((KA, FM), jnp.float32),
            pltpu.VMEM((KA, FM), jnp.float32),
            pltpu.VMEM((KA, FM), jnp.float32),
            pltpu.VMEM((KA * 16,), jnp.float32),
            pltpu.VMEM((KA, FM), jnp.float32),
            pltpu.VMEM((ZR, FM), jnp.float32),
            pltpu.VMEM_SHARED((NP, FM), jnp.float32),
            pltpu.SemaphoreType.DMA,
            pltpu.SemaphoreType.DMA,
            pltpu.SemaphoreType.DMA,
            pltpu.SemaphoreType.DMA,
        ],
    )
    return k(src_all, dst_all, als, ald, zeros)


def _edge_agg_body(src_hbm, dst_hbm, xw_hbm, ex_hbm, dinv_hbm, zeros_hbm,
                   y_out,
                   sbs0, sbd0, sms0, sms1, smd0, smd1,
                   xv0, xv1, dv0, dv1, exl_sb, cv, zv, y_sh,
                   sem_x0, sem_x1, sem_d0, sem_d1):
    c = lax.axis_index("c")
    s = lax.axis_index("s")
    wid = s * NC + c
    _zero_acc(y_sh, zeros_hbm, zv, s)
    plsc.subcore_barrier()

    base0 = wid * TA
    sm = ((sms0, smd0), (sms1, smd1))
    gb = ((xv0, dv0, sem_x0, sem_d0), (xv1, dv1, sem_x1, sem_d1))

    def load_sb(sb):
        off = base0 + sb * SBE
        pltpu.sync_copy(src_hbm.at[pl.ds(off, SBE + KB)], sbs0)
        pltpu.sync_copy(dst_hbm.at[pl.ds(off, SBE + KB)], sbd0)

    def prep(j, b):
        sm[b][0][...] = sbs0[pl.ds(j * KB, 16)]
        sm[b][1][...] = sbd0[pl.ds(j * KB, 16)]
        xv, dv, sx, sd = gb[b]
        pltpu.async_copy(xw_hbm.at[sm[b][0]], xv, sx)
        pltpu.async_copy(dinv_hbm.at[sm[b][1]], dv, sd)

    def step(j, b, do_prep=True):
        xv, dv, sx, sd = gb[b]
        pltpu.make_async_copy(xw_hbm.at[sm[b][0]], xv, sx).wait()
        pltpu.make_async_copy(dinv_hbm.at[sm[b][1]], dv, sd).wait()
        if do_prep:
            prep(j + 1, 1 - b)

        def edge(i, _):
            wrow = (exl_sb[pl.ds((j * KB + i) * 16, 16)]
                    * dv[i, pl.ds(0, 16)])
            acc = [jnp.zeros((16,), jnp.float32) for _ in range(FM // 16)]
            for h in range(H):
                w = jnp.full((16,), wrow[h])
                for jf in range(FM // 16):
                    acc[jf] = acc[jf] + w * xv[i, pl.ds(h * FM + jf * 16, 16)]
            for jf in range(FM // 16):
                cv[i, pl.ds(jf * 16, 16)] = acc[jf]
            return 0

        lax.fori_loop(0, KB, edge, 0)
        pltpu.sync_copy(cv, y_sh.at[sm[b][1]], add=True)

    def load_exl(sb):
        pltpu.sync_copy(
            ex_hbm.at[pl.ds((base0 + sb * SBE) * 16, SBE * 16)], exl_sb)

    load_sb(0)
    prep(0, 0)

    def outer(sb, _):
        load_exl(sb)

        def inner(jj, _):
            for b in range(2):
                step(2 * jj + b, b)
            return 0

        lax.fori_loop(0, SB_B // 2, inner, 0)
        load_sb(sb + 1)
        return 0

    lax.fori_loop(0, NSB - 1, outer, 0)
    load_exl(NSB - 1)

    def inner_f(jj, _):
        for b in range(2):
            step(2 * jj + b, b)
        return 0

    lax.fori_loop(0, SB_B // 2 - 1, inner_f, 0)
    step(SB_B - 2, 0)
    step(SB_B - 1, 1, do_prep=False)
    plsc.subcore_barrier()
    _drain_acc(y_sh, y_out, zv, c, s)


def _sc_edge_agg(src_all, dst_all, xw, ex, dinv, zeros):
    k = pl.kernel(
        _edge_agg_body,
        out_type=jax.ShapeDtypeStruct((NC * NP, FM), jnp.float32),
        mesh=_MESH,
        scratch_types=[
            pltpu.VMEM((SBE + KB,), jnp.int32),
            pltpu.VMEM((SBE + KB,), jnp.int32),
            pltpu.VMEM((KB,), jnp.int32),
            pltpu.VMEM((KB,), jnp.int32),
            pltpu.VMEM((KB,), jnp.int32),
            pltpu.VMEM((KB,), jnp.int32),
            pltpu.VMEM((KB, H * FM), jnp.float32),
            pltpu.VMEM((KB, H * FM), jnp.float32),
            pltpu.VMEM((KB, FM), jnp.float32),
            pltpu.VMEM((KB, FM), jnp.float32),
            pltpu.VMEM((SBE * 16,), jnp.float32),
            pltpu.VMEM((KB, FM), jnp.float32),
            pltpu.VMEM((ZR, FM), jnp.float32),
            pltpu.VMEM_SHARED((NP, FM), jnp.float32),
            pltpu.SemaphoreType.DMA,
            pltpu.SemaphoreType.DMA,
            pltpu.SemaphoreType.DMA,
            pltpu.SemaphoreType.DMA,
        ],
    )
    return k(src_all, dst_all, xw, ex, dinv, zeros)


def _gat_layer(src_all, dst_all, xw, als, ald, zeros):
    ex, denom = _sc_edge_att(src_all, dst_all, als, ald, zeros)
    dinv = _tc_dinv(denom)
    return _sc_edge_agg(src_all, dst_all, xw, ex, dinv, zeros)


def kernel(x, edge_index, W1, a_src1, a_dst1, b1, W2, a_src2, a_dst2, b2,
           conv_w, conv_b):
    x_pad = jnp.zeros((NP, FM), jnp.float32).at[:N].set(x)
    loops = jnp.arange(N, dtype=jnp.int32)
    dummy = jnp.full((EP + SBE - E_REAL,), N, dtype=jnp.int32)
    src_all = jnp.concatenate([edge_index[0], loops, dummy])
    dst_all = jnp.concatenate([edge_index[1], loops, dummy])
    a_src1_f = a_src1.reshape(1, H * FM)
    a_dst1_f = a_dst1.reshape(1, H * FM)
    a_src2_f = a_src2.reshape(1, H * FM)
    a_dst2_f = a_dst2.reshape(1, H * FM)
    b1_row = b1.reshape(1, FM)
    b2_row = b2.reshape(1, FM)
    c1t = conv_w[:, 0, :, 0].T
    c2t = conv_w[:, 1, :, 0].T
    conv_b_row = conv_b.reshape(1, OUT_CH)
    zeros = jnp.zeros((ZR, FM), jnp.float32)

    xw1, als1, ald1 = _tc_linear1(x_pad, W1, a_src1_f, a_dst1_f)
    y1part = _gat_layer(src_all, dst_all, xw1, als1, ald1, zeros)

    x1, xw2, als2, ald2 = _tc_linear2(y1part, b1_row, W2, a_src2_f, a_dst2_f)
    y2part = _gat_layer(src_all, dst_all, xw2, als2, ald2, zeros)

    xo = _tc_final(y2part, b2_row, x1, c1t, c2t, conv_b_row)
    circ_pad = xo[:512]
    mirna_pad = xo[N_CIRC:N_CIRC + 9600]
    scores_full = _tc_scores(circ_pad, mirna_pad)

    circ = xo[:N_CIRC]
    mirna = xo[N_CIRC:N]
    scores = scores_full[:N_CIRC, :N - N_CIRC]
    return scores, circ, mirna


# speedup vs baseline: 1.2606x; 1.0779x over previous
import jax
import jax.numpy as jnp
from jax import lax
from jax.experimental import pallas as pl
from jax.experimental.pallas import tpu as pltpu
from jax.experimental.pallas import tpu_sc as plsc

N = 10000
FM = 128
H = 8
E = 320000
OUT_CH = 128
N_CIRC = 504

NP = 10240
NC, NS = 2, 16
NW = NC * NS
E_REAL = E + N
TA = 10368
EP = TA * NW
KA = 64
KB = 16
ZR = 32
ROWS_W = NP // NS

_MESH = plsc.VectorSubcoreMesh(core_axis_name="c", subcore_axis_name="s")


def _linear_body(xb, w_ref, asrc_ref, adst_ref, xw_ref, als_ref, ald_ref):
    xw = jnp.dot(xb, w_ref[...], preferred_element_type=jnp.float32)
    xw_ref[...] = xw
    row = lax.broadcasted_iota(jnp.int32, (H * FM, FM), 0) // FM
    col = lax.broadcasted_iota(jnp.int32, (H * FM, FM), 1)
    msk = (row == col).astype(jnp.float32)
    amat_s = jnp.reshape(asrc_ref[...], (H * FM, 1)) * msk
    amat_d = jnp.reshape(adst_ref[...], (H * FM, 1)) * msk
    als_ref[...] = jnp.dot(xw, amat_s, preferred_element_type=jnp.float32)
    ald_ref[...] = jnp.dot(xw, amat_d, preferred_element_type=jnp.float32)


def _linear1_kernel(x_ref, w_ref, asrc_ref, adst_ref, xw_ref, als_ref, ald_ref):
    _linear_body(x_ref[...], w_ref, asrc_ref, adst_ref, xw_ref, als_ref, ald_ref)


def _linear2_kernel(y0_ref, y1_ref, b_ref, w_ref, asrc_ref, adst_ref,
                    x_out_ref, xw_ref, als_ref, ald_ref):
    xb = jnp.maximum(y0_ref[...] + y1_ref[...] + b_ref[...], 0.0)
    x_out_ref[...] = xb
    _linear_body(xb, w_ref, asrc_ref, adst_ref, xw_ref, als_ref, ald_ref)


_BM = 512


def _tc_linear1(x_pad, W, a_src_f, a_dst_f):
    grid = NP // _BM
    return pl.pallas_call(
        _linear1_kernel,
        grid=(grid,),
        in_specs=[
            pl.BlockSpec((_BM, FM), lambda i: (i, 0)),
            pl.BlockSpec((FM, H * FM), lambda i: (0, 0)),
            pl.BlockSpec((1, H * FM), lambda i: (0, 0)),
            pl.BlockSpec((1, H * FM), lambda i: (0, 0)),
        ],
        out_specs=[
            pl.BlockSpec((_BM, H * FM), lambda i: (i, 0)),
            pl.BlockSpec((_BM, FM), lambda i: (i, 0)),
            pl.BlockSpec((_BM, FM), lambda i: (i, 0)),
        ],
        out_shape=[
            jax.ShapeDtypeStruct((NP, H * FM), jnp.float32),
            jax.ShapeDtypeStruct((NP, FM), jnp.float32),
            jax.ShapeDtypeStruct((NP, FM), jnp.float32),
        ],
    )(x_pad, W, a_src_f, a_dst_f)


def _tc_linear2(ypart, b, W, a_src_f, a_dst_f):
    grid = NP // _BM
    return pl.pallas_call(
        _linear2_kernel,
        grid=(grid,),
        in_specs=[
            pl.BlockSpec((_BM, FM), lambda i: (i, 0)),
            pl.BlockSpec((_BM, FM), lambda i: (i + NP // _BM, 0)),
            pl.BlockSpec((1, FM), lambda i: (0, 0)),
            pl.BlockSpec((FM, H * FM), lambda i: (0, 0)),
            pl.BlockSpec((1, H * FM), lambda i: (0, 0)),
            pl.BlockSpec((1, H * FM), lambda i: (0, 0)),
        ],
        out_specs=[
            pl.BlockSpec((_BM, FM), lambda i: (i, 0)),
            pl.BlockSpec((_BM, H * FM), lambda i: (i, 0)),
            pl.BlockSpec((_BM, FM), lambda i: (i, 0)),
            pl.BlockSpec((_BM, FM), lambda i: (i, 0)),
        ],
        out_shape=[
            jax.ShapeDtypeStruct((NP, FM), jnp.float32),
            jax.ShapeDtypeStruct((NP, H * FM), jnp.float32),
            jax.ShapeDtypeStruct((NP, FM), jnp.float32),
            jax.ShapeDtypeStruct((NP, FM), jnp.float32),
        ],
    )(ypart, ypart, b, W, a_src_f, a_dst_f)


def _dinv_kernel(d0_ref, d1_ref, out_ref):
    out_ref[...] = 1.0 / (H * (d0_ref[...] + d1_ref[...]) + H * 1e-16)


def _tc_dinv(denom_flat):
    blk = 1024
    return pl.pallas_call(
        _dinv_kernel,
        grid=(NP // blk,),
        in_specs=[
            pl.BlockSpec((blk, FM), lambda i: (i, 0)),
            pl.BlockSpec((blk, FM), lambda i: (i + NP // blk, 0)),
        ],
        out_specs=pl.BlockSpec((blk, FM), lambda i: (i, 0)),
        out_shape=jax.ShapeDtypeStruct((NP, FM), jnp.float32),
    )(denom_flat, denom_flat)


def _final_kernel(y0_ref, y1_ref, b2_ref, x1_ref, c1_ref, c2_ref, cb_ref, xo_ref):
    x2 = jnp.maximum(y0_ref[...] + y1_ref[...] + b2_ref[...], 0.0)
    xo_ref[...] = (
        jnp.dot(x1_ref[...], c1_ref[...], preferred_element_type=jnp.float32)
        + jnp.dot(x2, c2_ref[...], preferred_element_type=jnp.float32)
        + cb_ref[...]
    )


def _tc_final(ypart2, b2, x1, c1t, c2t, conv_b_row):
    grid = NP // _BM
    return pl.pallas_call(
        _final_kernel,
        grid=(grid,),
        in_specs=[
            pl.BlockSpec((_BM, FM), lambda i: (i, 0)),
            pl.BlockSpec((_BM, FM), lambda i: (i + NP // _BM, 0)),
            pl.BlockSpec((1, FM), lambda i: (0, 0)),
            pl.BlockSpec((_BM, FM), lambda i: (i, 0)),
            pl.BlockSpec((FM, OUT_CH), lambda i: (0, 0)),
            pl.BlockSpec((FM, OUT_CH), lambda i: (0, 0)),
            pl.BlockSpec((1, OUT_CH), lambda i: (0, 0)),
        ],
        out_specs=pl.BlockSpec((_BM, OUT_CH), lambda i: (i, 0)),
        out_shape=jax.ShapeDtypeStruct((NP, OUT_CH), jnp.float32),
    )(ypart2, ypart2, b2, x1, c1t, c2t, conv_b_row)


def _scores_kernel(a_ref, b_ref, out_ref):
    out_ref[...] = lax.dot_general(
        a_ref[...], b_ref[...], (((1,), (1,)), ((), ())),
        preferred_element_type=jnp.float32)


def _tc_scores(circ_pad, mirna_pad):
    bn = 640
    nb = mirna_pad.shape[0] // bn
    return pl.pallas_call(
        _scores_kernel,
        grid=(nb,),
        in_specs=[
            pl.BlockSpec((512, FM), lambda i: (0, 0)),
            pl.BlockSpec((bn, FM), lambda i: (i, 0)),
        ],
        out_specs=pl.BlockSpec((512, bn), lambda i: (0, i)),
        out_shape=jax.ShapeDtypeStruct((512, nb * bn), jnp.float32),
    )(circ_pad, mirna_pad)


def _zero_acc(acc_sh, zeros_hbm, zv, s):
    pltpu.sync_copy(zeros_hbm, zv)

    def z(i, _):
        pltpu.sync_copy(zv, acc_sh.at[pl.ds(s * ROWS_W + i * ZR, ZR)])
        return 0

    lax.fori_loop(0, ROWS_W // ZR, z, 0)


def _drain_acc(acc_sh, out_hbm, zv, c, s):
    def d(i, _):
        pltpu.sync_copy(acc_sh.at[pl.ds(s * ROWS_W + i * ZR, ZR)], zv)
        pltpu.sync_copy(zv, out_hbm.at[pl.ds(c * NP + s * ROWS_W + i * ZR, ZR)])
        return 0

    lax.fori_loop(0, ROWS_W // ZR, d, 0)


def _edge_att_body(src_hbm, dst_hbm, als_hbm, ald_hbm, zeros_hbm,
                   ex_out, denom_out,
                   sidx, didx, av, bv, exl, exs, zv, denom_sh, sem_a, sem_b):
    c = lax.axis_index("c")
    s = lax.axis_index("s")
    wid = s * NC + c
    _zero_acc(denom_sh, zeros_hbm, zv, s)
    pltpu.sync_copy(zeros_hbm, exs.at[pl.ds(0, ZR)])
    pltpu.sync_copy(zeros_hbm, exs.at[pl.ds(ZR, ZR)])
    plsc.subcore_barrier()

    base0 = wid * TA

    def batch(bi, _):
        base = base0 + bi * KA
        pltpu.sync_copy(src_hbm.at[pl.ds(base, KA)], sidx)
        pltpu.sync_copy(dst_hbm.at[pl.ds(base, KA)], didx)
        cp_a = pltpu.async_copy(als_hbm.at[sidx], av, sem_a)
        cp_b = pltpu.async_copy(ald_hbm.at[didx], bv, sem_b)
        cp_a.wait()
        cp_b.wait()

        def row(i, _):
            a = av[i, pl.ds(0, 16)] + bv[i, pl.ds(0, 16)]
            al = jnp.maximum(a, 0.2 * a)
            e = jnp.exp(al)
            exl[pl.ds(i * 16, 16)] = e
            exs[i, pl.ds(0, 16)] = e
            return 0

        lax.fori_loop(0, KA, row, 0)
        pltpu.sync_copy(exl, ex_out.at[pl.ds(base * 16, KA * 16)])
        pltpu.sync_copy(exs, denom_sh.at[didx], add=True)
        return 0

    lax.fori_loop(0, TA // KA, batch, 0)
    plsc.subcore_barrier()
    _drain_acc(denom_sh, denom_out, zv, c, s)


def _sc_edge_att(src_all, dst_all, als, ald, zeros):
    k = pl.kernel(
        _edge_att_body,
        out_type=[
            jax.ShapeDtypeStruct((EP * 16,), jnp.float32),
            jax.ShapeDtypeStruct((NC * NP, FM), jnp.float32),
        ],
        mesh=_MESH,
        scratch_types=[
            pltpu.VMEM((KA,), jnp.int32),
            pltpu.VMEM((KA,), jnp.int32),
            pltpu.VMEM((KA, FM), jnp.float32),
            pltpu.VMEM((KA, FM), jnp.float32),
            pltpu.VMEM((KA * 16,), jnp.float32),
            pltpu.VMEM((KA, FM), jnp.float32),
            pltpu.VMEM((ZR, FM), jnp.float32),
            pltpu.VMEM_SHARED((NP, FM), jnp.float32),
            pltpu.SemaphoreType.DMA,
            pltpu.SemaphoreType.DMA,
        ],
    )
    return k(src_all, dst_all, als, ald, zeros)


def _edge_agg_body(src_hbm, dst_hbm, xw_hbm, ex_hbm, dinv_hbm, zeros_hbm,
                   y_out,
                   sidx0, sidx1, didx0, didx1, xv0, xv1, exl0, exl1,
                   dv0, dv1, cv, zv, y_sh,
                   sem_x0, sem_x1, sem_d0, sem_d1, sem_e0, sem_e1):
    c = lax.axis_index("c")
    s = lax.axis_index("s")
    wid = s * NC + c
    _zero_acc(y_sh, zeros_hbm, zv, s)
    plsc.subcore_barrier()

    base0 = wid * TA
    nb = TA // KB
    bufs = ((sidx0, didx0, xv0, dv0, exl0, sem_x0, sem_d0, sem_e0),
            (sidx1, didx1, xv1, dv1, exl1, sem_x1, sem_d1, sem_e1))

    def issue(bi, b):
        si, di, xv, dv, exl, sx, sd, se = bufs[b]
        base = base0 + bi * KB
        pltpu.sync_copy(src_hbm.at[pl.ds(base, KB)], si)
        pltpu.sync_copy(dst_hbm.at[pl.ds(base, KB)], di)
        pltpu.async_copy(xw_hbm.at[si], xv, sx)
        pltpu.async_copy(dinv_hbm.at[di], dv, sd)
        pltpu.async_copy(ex_hbm.at[pl.ds(base * 16, KB * 16)], exl, se)

    def step(bi, b):
        si, di, xv, dv, exl, sx, sd, se = bufs[b]
        pltpu.make_async_copy(xw_hbm.at[si], xv, sx).wait()
        pltpu.make_async_copy(dinv_hbm.at[di], dv, sd).wait()
        pltpu.make_async_copy(ex_hbm.at[pl.ds(0, KB * 16)], exl, se).wait()

        @pl.when(bi + 1 < nb)
        def _():
            issue(bi + 1, 1 - b)

        def edge(i, _):
            wrow = exl[pl.ds(i * 16, 16)] * dv[i, pl.ds(0, 16)]
            acc = [jnp.zeros((16,), jnp.float32) for _ in range(FM // 16)]
            for h in range(H):
                w = jnp.full((16,), wrow[h])
                for jf in range(FM // 16):
                    acc[jf] = acc[jf] + w * xv[i, pl.ds(h * FM + jf * 16, 16)]
            for jf in range(FM // 16):
                cv[i, pl.ds(jf * 16, 16)] = acc[jf]
            return 0

        lax.fori_loop(0, KB, edge, 0)
        pltpu.sync_copy(cv, y_sh.at[di], add=True)

    issue(0, 0)

    def pair(g, _):
        for b in range(2):
            step(2 * g + b, b)
        return 0

    lax.fori_loop(0, nb // 2, pair, 0)
    plsc.subcore_barrier()
    _drain_acc(y_sh, y_out, zv, c, s)


def _sc_edge_agg(src_all, dst_all, xw, ex, dinv, zeros):
    k = pl.kernel(
        _edge_agg_body,
        out_type=jax.ShapeDtypeStruct((NC * NP, FM), jnp.float32),
        mesh=_MESH,
        scratch_types=[
            pltpu.VMEM((KB,), jnp.int32),
            pltpu.VMEM((KB,), jnp.int32),
            pltpu.VMEM((KB,), jnp.int32),
            pltpu.VMEM((KB,), jnp.int32),
            pltpu.VMEM((KB, H * FM), jnp.float32),
            pltpu.VMEM((KB, H * FM), jnp.float32),
            pltpu.VMEM((KB * 16,), jnp.float32),
            pltpu.VMEM((KB * 16,), jnp.float32),
            pltpu.VMEM((KB, FM), jnp.float32),
            pltpu.VMEM((KB, FM), jnp.float32),
            pltpu.VMEM((KB, FM), jnp.float32),
            pltpu.VMEM((ZR, FM), jnp.float32),
            pltpu.VMEM_SHARED((NP, FM), jnp.float32),
            pltpu.SemaphoreType.DMA,
            pltpu.SemaphoreType.DMA,
            pltpu.SemaphoreType.DMA,
            pltpu.SemaphoreType.DMA,
            pltpu.SemaphoreType.DMA,
            pltpu.SemaphoreType.DMA,
        ],
    )
    return k(src_all, dst_all, xw, ex, dinv, zeros)


def _gat_layer(src_all, dst_all, xw, als, ald, zeros):
    ex, denom = _sc_edge_att(src_all, dst_all, als, ald, zeros)
    dinv = _tc_dinv(denom)
    return _sc_edge_agg(src_all, dst_all, xw, ex, dinv, zeros)


def kernel(x, edge_index, W1, a_src1, a_dst1, b1, W2, a_src2, a_dst2, b2,
           conv_w, conv_b):
    x_pad = jnp.zeros((NP, FM), jnp.float32).at[:N].set(x)
    loops = jnp.arange(N, dtype=jnp.int32)
    dummy = jnp.full((EP - E_REAL,), N, dtype=jnp.int32)
    src_all = jnp.concatenate([edge_index[0], loops, dummy])
    dst_all = jnp.concatenate([edge_index[1], loops, dummy])
    a_src1_f = a_src1.reshape(1, H * FM)
    a_dst1_f = a_dst1.reshape(1, H * FM)
    a_src2_f = a_src2.reshape(1, H * FM)
    a_dst2_f = a_dst2.reshape(1, H * FM)
    b1_row = b1.reshape(1, FM)
    b2_row = b2.reshape(1, FM)
    c1t = conv_w[:, 0, :, 0].T
    c2t = conv_w[:, 1, :, 0].T
    conv_b_row = conv_b.reshape(1, OUT_CH)
    zeros = jnp.zeros((ZR, FM), jnp.float32)

    xw1, als1, ald1 = _tc_linear1(x_pad, W1, a_src1_f, a_dst1_f)
    y1part = _gat_layer(src_all, dst_all, xw1, als1, ald1, zeros)

    x1, xw2, als2, ald2 = _tc_linear2(y1part, b1_row, W2, a_src2_f, a_dst2_f)
    y2part = _gat_layer(src_all, dst_all, xw2, als2, ald2, zeros)

    xo = _tc_final(y2part, b2_row, x1, c1t, c2t, conv_b_row)
    circ_pad = xo[:512]
    mirna_pad = xo[N_CIRC:N_CIRC + 9600]
    scores_full = _tc_scores(circ_pad, mirna_pad)

    circ = xo[:N_CIRC]
    mirna = xo[N_CIRC:N]
    scores = scores_full[:N_CIRC, :N - N_CIRC]
    return scores, circ, mirna


# speedup vs baseline: 1.3011x; 1.0321x over previous
import jax
import jax.numpy as jnp
from jax import lax
from jax.experimental import pallas as pl
from jax.experimental.pallas import tpu as pltpu
from jax.experimental.pallas import tpu_sc as plsc

N = 10000
FM = 128
H = 8
E = 320000
OUT_CH = 128
N_CIRC = 504

NP = 10240
NC, NS = 2, 16
NW = NC * NS
E_REAL = E + N
TA = 10368
EP = TA * NW
KA = 96
KB = 16
ZR = 32
ROWS_W = NP // NS

_MESH = plsc.VectorSubcoreMesh(core_axis_name="c", subcore_axis_name="s")


def _linear_body(xb, w_ref, asrc_ref, adst_ref, xw_ref, als_ref, ald_ref):
    xw = jnp.dot(xb, w_ref[...], preferred_element_type=jnp.float32)
    xw_ref[...] = xw
    row = lax.broadcasted_iota(jnp.int32, (H * FM, FM), 0) // FM
    col = lax.broadcasted_iota(jnp.int32, (H * FM, FM), 1)
    msk = (row == col).astype(jnp.float32)
    amat_s = jnp.reshape(asrc_ref[...], (H * FM, 1)) * msk
    amat_d = jnp.reshape(adst_ref[...], (H * FM, 1)) * msk
    als_ref[...] = jnp.dot(xw, amat_s, preferred_element_type=jnp.float32)
    ald_ref[...] = jnp.dot(xw, amat_d, preferred_element_type=jnp.float32)


def _linear1_kernel(x_ref, w_ref, asrc_ref, adst_ref, xw_ref, als_ref, ald_ref):
    _linear_body(x_ref[...], w_ref, asrc_ref, adst_ref, xw_ref, als_ref, ald_ref)


def _linear2_kernel(y0_ref, y1_ref, b_ref, w_ref, asrc_ref, adst_ref,
                    x_out_ref, xw_ref, als_ref, ald_ref):
    xb = jnp.maximum(y0_ref[...] + y1_ref[...] + b_ref[...], 0.0)
    x_out_ref[...] = xb
    _linear_body(xb, w_ref, asrc_ref, adst_ref, xw_ref, als_ref, ald_ref)


_BM = 512


def _tc_linear1(x_pad, W, a_src_f, a_dst_f):
    grid = NP // _BM
    return pl.pallas_call(
        _linear1_kernel,
        grid=(grid,),
        in_specs=[
            pl.BlockSpec((_BM, FM), lambda i: (i, 0)),
            pl.BlockSpec((FM, H * FM), lambda i: (0, 0)),
            pl.BlockSpec((1, H * FM), lambda i: (0, 0)),
            pl.BlockSpec((1, H * FM), lambda i: (0, 0)),
        ],
        out_specs=[
            pl.BlockSpec((_BM, H * FM), lambda i: (i, 0)),
            pl.BlockSpec((_BM, FM), lambda i: (i, 0)),
            pl.BlockSpec((_BM, FM), lambda i: (i, 0)),
        ],
        out_shape=[
            jax.ShapeDtypeStruct((NP, H * FM), jnp.float32),
            jax.ShapeDtypeStruct((NP, FM), jnp.float32),
            jax.ShapeDtypeStruct((NP, FM), jnp.float32),
        ],
    )(x_pad, W, a_src_f, a_dst_f)


def _tc_linear2(ypart, b, W, a_src_f, a_dst_f):
    grid = NP // _BM
    return pl.pallas_call(
        _linear2_kernel,
        grid=(grid,),
        in_specs=[
            pl.BlockSpec((_BM, FM), lambda i: (i, 0)),
            pl.BlockSpec((_BM, FM), lambda i: (i + NP // _BM, 0)),
            pl.BlockSpec((1, FM), lambda i: (0, 0)),
            pl.BlockSpec((FM, H * FM), lambda i: (0, 0)),
            pl.BlockSpec((1, H * FM), lambda i: (0, 0)),
            pl.BlockSpec((1, H * FM), lambda i: (0, 0)),
        ],
        out_specs=[
            pl.BlockSpec((_BM, FM), lambda i: (i, 0)),
            pl.BlockSpec((_BM, H * FM), lambda i: (i, 0)),
            pl.BlockSpec((_BM, FM), lambda i: (i, 0)),
            pl.BlockSpec((_BM, FM), lambda i: (i, 0)),
        ],
        out_shape=[
            jax.ShapeDtypeStruct((NP, FM), jnp.float32),
            jax.ShapeDtypeStruct((NP, H * FM), jnp.float32),
            jax.ShapeDtypeStruct((NP, FM), jnp.float32),
            jax.ShapeDtypeStruct((NP, FM), jnp.float32),
        ],
    )(ypart, ypart, b, W, a_src_f, a_dst_f)


def _dinv_kernel(d0_ref, d1_ref, out_ref):
    out_ref[...] = 1.0 / (H * (d0_ref[...] + d1_ref[...]) + H * 1e-16)


def _tc_dinv(denom_flat):
    blk = 1024
    return pl.pallas_call(
        _dinv_kernel,
        grid=(NP // blk,),
        in_specs=[
            pl.BlockSpec((blk, FM), lambda i: (i, 0)),
            pl.BlockSpec((blk, FM), lambda i: (i + NP // blk, 0)),
        ],
        out_specs=pl.BlockSpec((blk, FM), lambda i: (i, 0)),
        out_shape=jax.ShapeDtypeStruct((NP, FM), jnp.float32),
    )(denom_flat, denom_flat)


def _final_kernel(y0_ref, y1_ref, b2_ref, x1_ref, c1_ref, c2_ref, cb_ref, xo_ref):
    x2 = jnp.maximum(y0_ref[...] + y1_ref[...] + b2_ref[...], 0.0)
    xo_ref[...] = (
        jnp.dot(x1_ref[...], c1_ref[...], preferred_element_type=jnp.float32)
        + jnp.dot(x2, c2_ref[...], preferred_element_type=jnp.float32)
        + cb_ref[...]
    )


def _tc_final(ypart2, b2, x1, c1t, c2t, conv_b_row):
    grid = NP // _BM
    return pl.pallas_call(
        _final_kernel,
        grid=(grid,),
        in_specs=[
            pl.BlockSpec((_BM, FM), lambda i: (i, 0)),
            pl.BlockSpec((_BM, FM), lambda i: (i + NP // _BM, 0)),
            pl.BlockSpec((1, FM), lambda i: (0, 0)),
            pl.BlockSpec((_BM, FM), lambda i: (i, 0)),
            pl.BlockSpec((FM, OUT_CH), lambda i: (0, 0)),
            pl.BlockSpec((FM, OUT_CH), lambda i: (0, 0)),
            pl.BlockSpec((1, OUT_CH), lambda i: (0, 0)),
        ],
        out_specs=pl.BlockSpec((_BM, OUT_CH), lambda i: (i, 0)),
        out_shape=jax.ShapeDtypeStruct((NP, OUT_CH), jnp.float32),
    )(ypart2, ypart2, b2, x1, c1t, c2t, conv_b_row)


def _scores_kernel(a_ref, b_ref, out_ref):
    out_ref[...] = lax.dot_general(
        a_ref[...], b_ref[...], (((1,), (1,)), ((), ())),
        preferred_element_type=jnp.float32)


def _tc_scores(circ_pad, mirna_pad):
    bn = 640
    nb = mirna_pad.shape[0] // bn
    return pl.pallas_call(
        _scores_kernel,
        grid=(nb,),
        in_specs=[
            pl.BlockSpec((512, FM), lambda i: (0, 0)),
            pl.BlockSpec((bn, FM), lambda i: (i, 0)),
        ],
        out_specs=pl.BlockSpec((512, bn), lambda i: (0, i)),
        out_shape=jax.ShapeDtypeStruct((512, nb * bn), jnp.float32),
    )(circ_pad, mirna_pad)


def _zero_acc(acc_sh, zeros_hbm, zv, s):
    pltpu.sync_copy(zeros_hbm, zv)

    def z(i, _):
        pltpu.sync_copy(zv, acc_sh.at[pl.ds(s * ROWS_W + i * ZR, ZR)])
        return 0

    lax.fori_loop(0, ROWS_W // ZR, z, 0)


def _drain_acc(acc_sh, out_hbm, zv, c, s):
    def d(i, _):
        pltpu.sync_copy(acc_sh.at[pl.ds(s * ROWS_W + i * ZR, ZR)], zv)
        pltpu.sync_copy(zv, out_hbm.at[pl.ds(c * NP + s * ROWS_W + i * ZR, ZR)])
        return 0

    lax.fori_loop(0, ROWS_W // ZR, d, 0)


def _edge_att_body(src_hbm, dst_hbm, als_hbm, ald_hbm, zeros_hbm,
                   ex_out, denom_out,
                   sidx, didx, av, bv, exl, exs, zv, denom_sh, sem_a, sem_b):
    c = lax.axis_index("c")
    s = lax.axis_index("s")
    wid = s * NC + c
    _zero_acc(denom_sh, zeros_hbm, zv, s)
    for z in range(KA // ZR):
        pltpu.sync_copy(zeros_hbm, exs.at[pl.ds(z * ZR, ZR)])
    plsc.subcore_barrier()

    base0 = wid * TA

    def batch(bi, _):
        base = base0 + bi * KA
        pltpu.sync_copy(src_hbm.at[pl.ds(base, KA)], sidx)
        pltpu.sync_copy(dst_hbm.at[pl.ds(base, KA)], didx)
        cp_a = pltpu.async_copy(als_hbm.at[sidx], av, sem_a)
        cp_b = pltpu.async_copy(ald_hbm.at[didx], bv, sem_b)
        cp_a.wait()
        cp_b.wait()

        def row(i, _):
            a = av[i, pl.ds(0, 16)] + bv[i, pl.ds(0, 16)]
            al = jnp.maximum(a, 0.2 * a)
            e = jnp.exp(al)
            exl[pl.ds(i * 16, 16)] = e
            exs[i, pl.ds(0, 16)] = e
            return 0

        lax.fori_loop(0, KA, row, 0)
        pltpu.sync_copy(exl, ex_out.at[pl.ds(base * 16, KA * 16)])
        pltpu.sync_copy(exs, denom_sh.at[didx], add=True)
        return 0

    lax.fori_loop(0, TA // KA, batch, 0)
    plsc.subcore_barrier()
    _drain_acc(denom_sh, denom_out, zv, c, s)


def _sc_edge_att(src_all, dst_all, als, ald, zeros):
    k = pl.kernel(
        _edge_att_body,
        out_type=[
            jax.ShapeDtypeStruct((EP * 16,), jnp.float32),
            jax.ShapeDtypeStruct((NC * NP, FM), jnp.float32),
        ],
        mesh=_MESH,
        scratch_types=[
            pltpu.VMEM((KA,), jnp.int32),
            pltpu.VMEM((KA,), jnp.int32),
            pltpu.VMEM((KA, FM), jnp.float32),
            pltpu.VMEM((KA, FM), jnp.float32),
            pltpu.VMEM((KA * 16,), jnp.float32),
            pltpu.VMEM((KA, FM), jnp.float32),
            pltpu.VMEM((ZR, FM), jnp.float32),
            pltpu.VMEM_SHARED((NP, FM), jnp.float32),
            pltpu.SemaphoreType.DMA,
            pltpu.SemaphoreType.DMA,
        ],
    )
    return k(src_all, dst_all, als, ald, zeros)


def _edge_agg_body(src_hbm, dst_hbm, xw_hbm, ex_hbm, dinv_hbm, zeros_hbm,
                   y_out,
                   sidx0, sidx1, didx0, didx1, xv0, xv1, exl0, exl1,
                   dv0, dv1, cv, zv, y_sh,
                   sem_x0, sem_x1, sem_d0, sem_d1, sem_e0, sem_e1):
    c = lax.axis_index("c")
    s = lax.axis_index("s")
    wid = s * NC + c
    _zero_acc(y_sh, zeros_hbm, zv, s)
    plsc.subcore_barrier()

    base0 = wid * TA
    nb = TA // KB
    bufs = ((sidx0, didx0, xv0, dv0, exl0, sem_x0, sem_d0, sem_e0),
            (sidx1, didx1, xv1, dv1, exl1, sem_x1, sem_d1, sem_e1))

    def issue(bi, b):
        si, di, xv, dv, exl, sx, sd, se = bufs[b]
        base = base0 + bi * KB
        pltpu.sync_copy(src_hbm.at[pl.ds(base, KB)], si)
        pltpu.sync_copy(dst_hbm.at[pl.ds(base, KB)], di)
        pltpu.async_copy(xw_hbm.at[si], xv, sx)
        pltpu.async_copy(dinv_hbm.at[di], dv, sd)
        pltpu.async_copy(ex_hbm.at[pl.ds(base * 16, KB * 16)], exl, se)

    def step(bi, b):
        si, di, xv, dv, exl, sx, sd, se = bufs[b]
        pltpu.make_async_copy(xw_hbm.at[si], xv, sx).wait()
        pltpu.make_async_copy(dinv_hbm.at[di], dv, sd).wait()
        pltpu.make_async_copy(ex_hbm.at[pl.ds(0, KB * 16)], exl, se).wait()

        @pl.when(bi + 1 < nb)
        def _():
            issue(bi + 1, 1 - b)

        def edge(i, _):
            wrow = exl[pl.ds(i * 16, 16)] * dv[i, pl.ds(0, 16)]
            acc = [jnp.zeros((16,), jnp.float32) for _ in range(FM // 16)]
            for h in range(H):
                w = jnp.full((16,), wrow[h])
                for jf in range(FM // 16):
                    acc[jf] = acc[jf] + w * xv[i, pl.ds(h * FM + jf * 16, 16)]
            for jf in range(FM // 16):
                cv[i, pl.ds(jf * 16, 16)] = acc[jf]
            return 0

        lax.fori_loop(0, KB, edge, 0)
        pltpu.sync_copy(cv, y_sh.at[di], add=True)

    issue(0, 0)

    def pair(g, _):
        for b in range(2):
            step(2 * g + b, b)
        return 0

    lax.fori_loop(0, nb // 2, pair, 0)
    plsc.subcore_barrier()
    _drain_acc(y_sh, y_out, zv, c, s)


def _sc_edge_agg(src_all, dst_all, xw, ex, dinv, zeros):
    k = pl.kernel(
        _edge_agg_body,
        out_type=jax.ShapeDtypeStruct((NC * NP, FM), jnp.float32),
        mesh=_MESH,
        scratch_types=[
            pltpu.VMEM((KB,), jnp.int32),
            pltpu.VMEM((KB,), jnp.int32),
            pltpu.VMEM((KB,), jnp.int32),
            pltpu.VMEM((KB,), jnp.int32),
            pltpu.VMEM((KB, H * FM), jnp.float32),
            pltpu.VMEM((KB, H * FM), jnp.float32),
            pltpu.VMEM((KB * 16,), jnp.float32),
            pltpu.VMEM((KB * 16,), jnp.float32),
            pltpu.VMEM((KB, FM), jnp.float32),
            pltpu.VMEM((KB, FM), jnp.float32),
            pltpu.VMEM((KB, FM), jnp.float32),
            pltpu.VMEM((ZR, FM), jnp.float32),
            pltpu.VMEM_SHARED((NP, FM), jnp.float32),
            pltpu.SemaphoreType.DMA,
            pltpu.SemaphoreType.DMA,
            pltpu.SemaphoreType.DMA,
            pltpu.SemaphoreType.DMA,
            pltpu.SemaphoreType.DMA,
            pltpu.SemaphoreType.DMA,
        ],
    )
    return k(src_all, dst_all, xw, ex, dinv, zeros)


def _gat_layer(src_all, dst_all, xw, als, ald, zeros):
    ex, denom = _sc_edge_att(src_all, dst_all, als, ald, zeros)
    dinv = _tc_dinv(denom)
    return _sc_edge_agg(src_all, dst_all, xw, ex, dinv, zeros)


def kernel(x, edge_index, W1, a_src1, a_dst1, b1, W2, a_src2, a_dst2, b2,
           conv_w, conv_b):
    x_pad = jnp.zeros((NP, FM), jnp.float32).at[:N].set(x)
    loops = jnp.arange(N, dtype=jnp.int32)
    dummy = jnp.full((EP - E_REAL,), N, dtype=jnp.int32)
    src_all = jnp.concatenate([edge_index[0], loops, dummy])
    dst_all = jnp.concatenate([edge_index[1], loops, dummy])
    a_src1_f = a_src1.reshape(1, H * FM)
    a_dst1_f = a_dst1.reshape(1, H * FM)
    a_src2_f = a_src2.reshape(1, H * FM)
    a_dst2_f = a_dst2.reshape(1, H * FM)
    b1_row = b1.reshape(1, FM)
    b2_row = b2.reshape(1, FM)
    c1t = conv_w[:, 0, :, 0].T
    c2t = conv_w[:, 1, :, 0].T
    conv_b_row = conv_b.reshape(1, OUT_CH)
    zeros = jnp.zeros((ZR, FM), jnp.float32)

    xw1, als1, ald1 = _tc_linear1(x_pad, W1, a_src1_f, a_dst1_f)
    y1part = _gat_layer(src_all, dst_all, xw1, als1, ald1, zeros)

    x1, xw2, als2, ald2 = _tc_linear2(y1part, b1_row, W2, a_src2_f, a_dst2_f)
    y2part = _gat_layer(src_all, dst_all, xw2, als2, ald2, zeros)

    xo = _tc_final(y2part, b2_row, x1, c1t, c2t, conv_b_row)
    circ_pad = xo[:512]
    mirna_pad = xo[N_CIRC:N_CIRC + 9600]
    scores_full = _tc_scores(circ_pad, mirna_pad)

    circ = xo[:N_CIRC]
    mirna = xo[N_CIRC:N]
    scores = scores_full[:N_CIRC, :N - N_CIRC]
    return scores, circ, mirna
